# baseline (speedup 1.0000x reference)
import numpy as np
import jax
import jax.numpy as jnp
from jax import lax
from jax.experimental import pallas as pl
from jax.experimental.pallas import tpu as pltpu
from jax.experimental.pallas import tpu_sc as plsc

FEAT_STRIDE = 16
IM_H, IM_W = 800.0, 800.0
PRE_NMS_TOPN = 6000
POST_NMS_TOPN = 300
MIN_SIZE = 16.0
A_, H_, W_ = 9, 50, 50
N_REAL = A_ * H_ * W_
NS = 16
PER = 1408
N_PAD = NS * PER
VPT = PER // 16
NEGB = np.float32(-3.0e38)
VALID_T = np.float32(-2.0e38)
OUT_ROWS = POST_NMS_TOPN + 4


def _anchor_columns():
    scales = np.array([8.0, 16.0, 32.0])
    ratios = np.array([0.5, 1.0, 2.0])
    base = np.array([1.0, 1.0, 16.0, 16.0]) - 1
    w = base[2] - base[0] + 1
    h = base[3] - base[1] + 1
    x_ctr = base[0] + 0.5 * (w - 1)
    y_ctr = base[1] + 0.5 * (h - 1)
    size_ratios = w * h / ratios
    ws = np.round(np.sqrt(size_ratios))
    hs = np.round(ws * ratios)

    def mk(ws, hs, xc, yc):
        ws = ws[:, None]
        hs = hs[:, None]
        return np.hstack([xc - 0.5 * (ws - 1), yc - 0.5 * (hs - 1),
                          xc + 0.5 * (ws - 1), yc + 0.5 * (hs - 1)])

    ratio_anchors = mk(ws, hs, x_ctr, y_ctr)
    out = []
    for i in range(ratio_anchors.shape[0]):
        a = ratio_anchors[i]
        aw = a[2] - a[0] + 1
        ah = a[3] - a[1] + 1
        axc = a[0] + 0.5 * (aw - 1)
        ayc = a[1] + 0.5 * (ah - 1)
        out.append(mk(aw * scales, ah * scales, axc, ayc))
    base9 = np.vstack(out).astype(np.float32)

    shift = np.arange(W_, dtype=np.float32) * FEAT_STRIDE
    sx, sy = np.meshgrid(shift, shift)
    shifts = np.stack([sx.ravel(), sy.ravel(), sx.ravel(), sy.ravel()], 1)
    anch = (shifts[:, None, :] + base9[None, :, :]).reshape(-1, 4)
    anch = anch.astype(np.float32)
    aw = anch[:, 2] - anch[:, 0] + 1.0
    ah = anch[:, 3] - anch[:, 1] + 1.0
    acx = anch[:, 0] + 0.5 * aw
    acy = anch[:, 1] + 0.5 * ah
    pad = N_PAD - N_REAL
    cols = [np.pad(c.astype(np.float32), (0, pad)) for c in (aw, ah, acx, acy)]
    return cols


_AW, _AH, _ACX, _ACY = _anchor_columns()

_KEY_NEG1E9 = int(np.int32(np.array(-1e9, np.float32).view(np.int32))
                  ^ np.int32(0x7FFFFFFF))
_KEY_HI = 0x3F800000


def _body(bg_h, fg_h, dx_h, dy_h, dw_h, dh_h, aw_h, ah_h, acx_h, acy_h,
          out_h,
          bg, fg, dx, dy, dwv, dhv, awv, ahv, acxv, acyv,
          keys, scs, bx1, by1, bx2, by2, bar,
          cs, cx1, cy1, cx2, cy2, car, cgi,
          stf, sti, recv, cntv, rois, cnt_sh, rec_sh):
    sid = lax.axis_index("s")
    base = sid * PER
    io = lax.iota(jnp.int32, 16)
    zi = jnp.zeros((16,), jnp.int32)
    zf = jnp.zeros((16,), jnp.float32)

    sti[...] = zi
    pltpu.sync_copy(sti, cnt_sh.at[pl.ds(sid * 128, 16)])
    stf[...] = zf
    pltpu.sync_copy(stf, rec_sh.at[pl.ds(sid * 128, 16)])

    for h_ref, v_ref in ((bg_h, bg), (fg_h, fg), (dx_h, dx), (dy_h, dy),
                         (dw_h, dwv), (dh_h, dhv), (aw_h, awv), (ah_h, ahv),
                         (acx_h, acxv), (acy_h, acyv)):
        pltpu.sync_copy(h_ref.at[pl.ds(base, PER)], v_ref)

    def zr(r, c):
        rois[pl.ds(r * 16, 16)] = zf
        return c

    lax.fori_loop(0, OUT_ROWS, zr, 0)

    def dec(j, c):
        sl = pl.ds(j * 16, 16)
        b = bg[sl]
        f = fg[sl]
        m = jnp.maximum(b, f)
        eb = jnp.exp(b - m)
        ef = jnp.exp(f - m)
        p = ef / (eb + ef)
        wsa = awv[sl]
        hsa = ahv[sl]
        pcx = dx[sl] * wsa + acxv[sl]
        pcy = dy[sl] * hsa + acyv[sl]
        pw = jnp.exp(dwv[sl]) * wsa
        ph = jnp.exp(dhv[sl]) * hsa
        x1 = jnp.minimum(jnp.maximum(pcx - 0.5 * pw, 0.0), IM_W - 1.0)
        y1 = jnp.minimum(jnp.maximum(pcy - 0.5 * ph, 0.0), IM_H - 1.0)
        x2 = jnp.minimum(jnp.maximum(pcx + 0.5 * pw, 0.0), IM_W - 1.0)
        y2 = jnp.minimum(jnp.maximum(pcy + 0.5 * ph, 0.0), IM_H - 1.0)
        ws_ = x2 - x1 + 1.0
        hs_ = y2 - y1 + 1.0
        ok = (ws_ >= MIN_SIZE) & (hs_ >= MIN_SIZE)
        s = jnp.where(ok, p, jnp.float32(-1e9))
        gi = base + j * 16 + io
        s = jnp.where(gi < N_REAL, s, NEGB)
        ki = lax.bitcast_convert_type(s, jnp.int32)
        keys[sl] = jnp.where(ki >= 0, ki, ki ^ jnp.int32(0x7FFFFFFF))
        scs[sl] = s
        bx1[sl] = x1
        by1[sl] = y1
        bx2[sl] = x2
        by2[sl] = y2
        bar[sl] = ws_ * hs_
        return c

    lax.fori_loop(0, VPT, dec, 0)
    plsc.subcore_barrier()

    def exch_lane(c_local, stamp):
        sti[...] = jnp.where(io == 0, c_local, jnp.where(io == 1, stamp, 0))
        pltpu.sync_copy(sti, cnt_sh.at[pl.ds(sid * 128, 16)])

        def cond(ok):
            return jnp.logical_not(ok)

        def body(ok):
            pltpu.sync_copy(cnt_sh, cntv)
            stamps = plsc.load_gather(cntv, [io * 128 + 1])
            return jnp.sum(jnp.where(stamps == stamp, 1, 0)) == NS

        lax.while_loop(cond, body, jnp.bool_(False))
        plsc.subcore_barrier()
        return plsc.load_gather(cntv, [io * 128])

    def exch(c_local, stamp):
        return jnp.sum(exch_lane(c_local, stamp))

    def count_pred(pred):
        def cb(j, acc):
            return acc + jnp.where(pred(j), 1, 0)
        return jnp.sum(lax.fori_loop(0, VPT, cb, zi))

    def count_ge(mid):
        return count_pred(lambda j: keys[pl.ds(j * 16, 16)] >= mid)

    c_pos = exch(count_ge(jnp.int32(1)), jnp.int32(1))

    def bs1(i, lohi):
        lo, hi = lohi
        mid = lo + ((hi - lo + 1) >> 1)
        c = exch(count_ge(mid), i + 2)
        big = c >= PRE_NMS_TOPN
        return (jnp.where(big, mid, lo), jnp.where(big, hi, mid - 1))

    lo, _ = lax.fori_loop(0, 30, bs1, (jnp.int32(1), jnp.int32(_KEY_HI)))
    tkey = jnp.where(c_pos >= PRE_NMS_TOPN, lo, jnp.int32(_KEY_NEG1E9))

    c_gt = exch(count_pred(lambda j: keys[pl.ds(j * 16, 16)] > tkey),
                jnp.int32(33))

    def count_eq_le(jmax):
        def pred(j):
            k = keys[pl.ds(j * 16, 16)]
            gi = base + j * 16 + io
            return (k == tkey) & (gi <= jmax)
        return count_pred(pred)

    def bs2(i, lohi):
        lo2, hi2 = lohi
        mid = (lo2 + hi2) >> 1
        h = c_gt + exch(count_eq_le(mid), i + 34)
        ok = h >= PRE_NMS_TOPN
        return (jnp.where(ok, lo2, mid + 1), jnp.where(ok, mid, hi2))

    idx_t, _ = lax.fori_loop(0, 15, bs2, (jnp.int32(0), jnp.int32(N_PAD - 1)))

    @pl.when(sid == 0)
    def _():
        st = jnp.where(io == 0, c_pos.astype(jnp.float32), jnp.float32(0.0))
        st = jnp.where(io == 1, (tkey >> 12).astype(jnp.float32), st)
        st = jnp.where(io == 2, (tkey & 0xFFF).astype(jnp.float32), st)
        st = jnp.where(io == 3, c_gt.astype(jnp.float32), st)
        st = jnp.where(io == 4, idx_t.astype(jnp.float32), st)
        rois[pl.ds(300 * 16, 16)] = st

    def cmp_(j, off):
        sl = pl.ds(j * 16, 16)
        k = keys[sl]
        gi = base + j * 16 + io
        elig = (k > tkey) | ((k == tkey) & (gi <= idx_t))
        dsl = pl.ds(off, 16)
        plsc.store_compressed(cs.at[dsl], scs[sl], mask=elig)
        plsc.store_compressed(cx1.at[dsl], bx1[sl], mask=elig)
        plsc.store_compressed(cy1.at[dsl], by1[sl], mask=elig)
        plsc.store_compressed(cx2.at[dsl], bx2[sl], mask=elig)
        plsc.store_compressed(cy2.at[dsl], by2[sl], mask=elig)
        plsc.store_compressed(car.at[dsl], bar[sl], mask=elig)
        plsc.store_compressed(cgi.at[dsl], gi, mask=elig)
        return off + jnp.max(plsc.all_reduce_population_count(elig))

    m = lax.fori_loop(0, VPT, cmp_, jnp.int32(0))
    cs[pl.ds(m, 16)] = jnp.full((16,), NEGB, jnp.float32)
    mvr = (m + 15) >> 4

    def nms(k, c):
        stampf = (k + 1).astype(jnp.float32)

        def am(j, st):
            bs_, bj = st
            s = cs[pl.ds(j * 16, 16)]
            upd = s > bs_
            return (jnp.where(upd, s, bs_), jnp.where(upd, j, bj))

        bs_, bj = lax.fori_loop(
            0, mvr, am, (jnp.full((16,), NEGB, jnp.float32), zi))
        smax = jnp.max(bs_)
        lidx = jnp.min(jnp.where(bs_ == smax, bj * 16 + io,
                                 jnp.int32(1 << 30)))
        fiv = zi + lidx
        xv = plsc.load_gather(cx1, [fiv])
        yv = plsc.load_gather(cy1, [fiv])
        x2v = plsc.load_gather(cx2, [fiv])
        y2v = plsc.load_gather(cy2, [fiv])
        av = plsc.load_gather(car, [fiv])
        gixf = jnp.max(plsc.load_gather(cgi, [fiv])).astype(jnp.float32)
        rec = jnp.where(io == 0, smax, jnp.float32(0.0))
        rec = jnp.where(io == 1, gixf, rec)
        rec = jnp.where(io == 2, xv, rec)
        rec = jnp.where(io == 3, yv, rec)
        rec = jnp.where(io == 4, x2v, rec)
        rec = jnp.where(io == 5, y2v, rec)
        rec = jnp.where(io == 6, av, rec)
        rec = jnp.where(io == 7, stampf, rec)
        stf[...] = rec
        pltpu.sync_copy(stf, rec_sh.at[pl.ds(sid * 128, 16)])

        def cond(ok):
            return jnp.logical_not(ok)

        def body(ok):
            pltpu.sync_copy(rec_sh, recv)
            stamps = plsc.load_gather(recv, [io * 128 + 7])
            return jnp.sum(jnp.where(stamps == stampf, 1, 0)) == NS

        lax.while_loop(cond, body, jnp.bool_(False))
        plsc.subcore_barrier()

        sa = plsc.load_gather(recv, [io * 128])
        ga = plsc.load_gather(recv, [io * 128 + 1])
        gmax = jnp.max(sa)
        valid = gmax > VALID_T
        gwin = jnp.min(jnp.where(sa == gmax, ga, jnp.float32(1e18)))
        wslot = (gwin.astype(jnp.int32) // PER) * 128
        xw = plsc.load_gather(recv, [zi + wslot + 2])
        yw = plsc.load_gather(recv, [zi + wslot + 3])
        xw2 = plsc.load_gather(recv, [zi + wslot + 4])
        yw2 = plsc.load_gather(recv, [zi + wslot + 5])
        arw = plsc.load_gather(recv, [zi + wslot + 6])

        @pl.when(valid)
        def _():
            def up(j, cc):
                sl = pl.ds(j * 16, 16)
                xx1 = jnp.maximum(xw, cx1[sl])
                yy1 = jnp.maximum(yw, cy1[sl])
                xx2 = jnp.minimum(xw2, cx2[sl])
                yy2 = jnp.minimum(yw2, cy2[sl])
                iw = jnp.maximum(0.0, xx2 - xx1 + 1.0)
                ih = jnp.maximum(0.0, yy2 - yy1 + 1.0)
                inter = iw * ih
                supp = inter * jnp.float32(1.7) > \
                    jnp.float32(0.7) * (arw + car[sl])
                cs[sl] = jnp.where(supp, NEGB, cs[sl])
                return cc

            lax.fori_loop(0, mvr, up, 0)

        @pl.when(valid & (sid == 0))
        def _():
            row = jnp.where(io == 1, xw, jnp.float32(0.0))
            row = jnp.where(io == 2, yw, row)
            row = jnp.where(io == 3, xw2, row)
            row = jnp.where(io == 4, yw2, row)
            plsc.store_scatter(rois, [k * 16 + io], row, mask=io < 5)

        return c

    lax.fori_loop(0, POST_NMS_TOPN, nms, 0)

    @pl.when(sid == 0)
    def _():
        pltpu.sync_copy(rois, out_h)


def _run_full(rpn_cls_score, rpn_bbox_pred):
    cls2 = rpn_cls_score.reshape(-1, 2)
    d = rpn_bbox_pred.reshape(-1, 4)
    pad = N_PAD - N_REAL

    def padded(col):
        return jnp.pad(col, (0, pad))

    args = [padded(cls2[:, 0]), padded(cls2[:, 1]),
            padded(d[:, 0]), padded(d[:, 1]), padded(d[:, 2]),
            padded(d[:, 3]),
            jnp.asarray(_AW), jnp.asarray(_AH),
            jnp.asarray(_ACX), jnp.asarray(_ACY)]

    mesh = plsc.VectorSubcoreMesh(core_axis_name="c", subcore_axis_name="s",
                                  num_cores=1, num_subcores=NS)
    f32, i32 = jnp.float32, jnp.int32
    scratch = [
        pltpu.VMEM((PER,), f32),
        pltpu.VMEM((PER,), f32),
        pltpu.VMEM((PER,), f32),
        pltpu.VMEM((PER,), f32),
        pltpu.VMEM((PER,), f32),
        pltpu.VMEM((PER,), f32),
        pltpu.VMEM((PER,), f32),
        pltpu.VMEM((PER,), f32),
        pltpu.VMEM((PER,), f32),
        pltpu.VMEM((PER,), f32),
        pltpu.VMEM((PER,), i32),
        pltpu.VMEM((PER,), f32),
        pltpu.VMEM((PER,), f32),
        pltpu.VMEM((PER,), f32),
        pltpu.VMEM((PER,), f32),
        pltpu.VMEM((PER,), f32),
        pltpu.VMEM((PER,), f32),
        pltpu.VMEM((PER + 16,), f32),
        pltpu.VMEM((PER + 16,), f32),
        pltpu.VMEM((PER + 16,), f32),
        pltpu.VMEM((PER + 16,), f32),
        pltpu.VMEM((PER + 16,), f32),
        pltpu.VMEM((PER + 16,), f32),
        pltpu.VMEM((PER + 16,), i32),
        pltpu.VMEM((16,), f32),
        pltpu.VMEM((16,), i32),
        pltpu.VMEM((2048,), f32),
        pltpu.VMEM((2048,), i32),
        pltpu.VMEM((OUT_ROWS * 16,), f32),
        pltpu.VMEM_SHARED((2048,), i32),
        pltpu.VMEM_SHARED((2048,), f32),
    ]
    out = pl.kernel(
        _body,
        out_type=jax.ShapeDtypeStruct((OUT_ROWS * 16,), jnp.float32),
        mesh=mesh,
        scratch_types=scratch,
        compiler_params=pltpu.CompilerParams(needs_layout_passes=False),
    )(*args)
    return out.reshape(OUT_ROWS, 16)


def kernel(rpn_cls_score, rpn_bbox_pred):
    return _run_full(rpn_cls_score, rpn_bbox_pred)[:POST_NMS_TOPN, :5]


# speedup vs baseline: 1.0493x; 1.0493x over previous
import numpy as np
import jax
import jax.numpy as jnp
from jax import lax
from jax.experimental import pallas as pl
from jax.experimental.pallas import tpu as pltpu
from jax.experimental.pallas import tpu_sc as plsc

FEAT_STRIDE = 16
IM_H, IM_W = 800.0, 800.0
PRE_NMS_TOPN = 6000
POST_NMS_TOPN = 300
MIN_SIZE = 16.0
A_, H_, W_ = 9, 50, 50
N_REAL = A_ * H_ * W_
NS = 16
PER = 1408
N_PAD = NS * PER
VPT = PER // 16
NEGB = np.float32(-3.0e38)
VALID_T = np.float32(-2.0e38)
OUT_ROWS = POST_NMS_TOPN + 4


def _anchor_columns():
    scales = np.array([8.0, 16.0, 32.0])
    ratios = np.array([0.5, 1.0, 2.0])
    base = np.array([1.0, 1.0, 16.0, 16.0]) - 1
    w = base[2] - base[0] + 1
    h = base[3] - base[1] + 1
    x_ctr = base[0] + 0.5 * (w - 1)
    y_ctr = base[1] + 0.5 * (h - 1)
    size_ratios = w * h / ratios
    ws = np.round(np.sqrt(size_ratios))
    hs = np.round(ws * ratios)

    def mk(ws, hs, xc, yc):
        ws = ws[:, None]
        hs = hs[:, None]
        return np.hstack([xc - 0.5 * (ws - 1), yc - 0.5 * (hs - 1),
                          xc + 0.5 * (ws - 1), yc + 0.5 * (hs - 1)])

    ratio_anchors = mk(ws, hs, x_ctr, y_ctr)
    out = []
    for i in range(ratio_anchors.shape[0]):
        a = ratio_anchors[i]
        aw = a[2] - a[0] + 1
        ah = a[3] - a[1] + 1
        axc = a[0] + 0.5 * (aw - 1)
        ayc = a[1] + 0.5 * (ah - 1)
        out.append(mk(aw * scales, ah * scales, axc, ayc))
    base9 = np.vstack(out).astype(np.float32)

    shift = np.arange(W_, dtype=np.float32) * FEAT_STRIDE
    sx, sy = np.meshgrid(shift, shift)
    shifts = np.stack([sx.ravel(), sy.ravel(), sx.ravel(), sy.ravel()], 1)
    anch = (shifts[:, None, :] + base9[None, :, :]).reshape(-1, 4)
    anch = anch.astype(np.float32)
    aw = anch[:, 2] - anch[:, 0] + 1.0
    ah = anch[:, 3] - anch[:, 1] + 1.0
    acx = anch[:, 0] + 0.5 * aw
    acy = anch[:, 1] + 0.5 * ah
    pad = N_PAD - N_REAL
    cols = [np.pad(c.astype(np.float32), (0, pad)) for c in (aw, ah, acx, acy)]
    return cols


_AW, _AH, _ACX, _ACY = _anchor_columns()

_KEY_NEG1E9 = int(np.int32(np.array(-1e9, np.float32).view(np.int32))
                  ^ np.int32(0x7FFFFFFF))
_KEY_HI = 0x3F800000


def _body(bg_h, fg_h, dx_h, dy_h, dw_h, dh_h, aw_h, ah_h, acx_h, acy_h,
          out_h,
          bg, fg, dx, dy, dwv, dhv, awv, ahv, acxv, acyv,
          keys, scs, bx1, by1, bx2, by2, bar,
          cs, cx1, cy1, cx2, cy2, car, cgi,
          stf, sti, recv, cntv, rois, cnt_sh, rec_sh):
    sid = lax.axis_index("s")
    base = sid * PER
    io = lax.iota(jnp.int32, 16)
    zi = jnp.zeros((16,), jnp.int32)
    zf = jnp.zeros((16,), jnp.float32)

    sti[...] = zi
    pltpu.sync_copy(sti, cnt_sh.at[pl.ds(sid * 128, 16)])
    stf[...] = zf
    pltpu.sync_copy(stf, rec_sh.at[pl.ds(sid * 128, 16)])

    for h_ref, v_ref in ((bg_h, bg), (fg_h, fg), (dx_h, dx), (dy_h, dy),
                         (dw_h, dwv), (dh_h, dhv), (aw_h, awv), (ah_h, ahv),
                         (acx_h, acxv), (acy_h, acyv)):
        pltpu.sync_copy(h_ref.at[pl.ds(base, PER)], v_ref)

    def zr(r, c):
        rois[pl.ds(r * 16, 16)] = zf
        return c

    lax.fori_loop(0, OUT_ROWS, zr, 0)

    def dec(j, c):
        sl = pl.ds(j * 16, 16)
        b = bg[sl]
        f = fg[sl]
        m = jnp.maximum(b, f)
        eb = jnp.exp(b - m)
        ef = jnp.exp(f - m)
        p = ef / (eb + ef)
        wsa = awv[sl]
        hsa = ahv[sl]
        pcx = dx[sl] * wsa + acxv[sl]
        pcy = dy[sl] * hsa + acyv[sl]
        pw = jnp.exp(dwv[sl]) * wsa
        ph = jnp.exp(dhv[sl]) * hsa
        x1 = jnp.minimum(jnp.maximum(pcx - 0.5 * pw, 0.0), IM_W - 1.0)
        y1 = jnp.minimum(jnp.maximum(pcy - 0.5 * ph, 0.0), IM_H - 1.0)
        x2 = jnp.minimum(jnp.maximum(pcx + 0.5 * pw, 0.0), IM_W - 1.0)
        y2 = jnp.minimum(jnp.maximum(pcy + 0.5 * ph, 0.0), IM_H - 1.0)
        ws_ = x2 - x1 + 1.0
        hs_ = y2 - y1 + 1.0
        ok = (ws_ >= MIN_SIZE) & (hs_ >= MIN_SIZE)
        s = jnp.where(ok, p, jnp.float32(-1e9))
        gi = base + j * 16 + io
        s = jnp.where(gi < N_REAL, s, NEGB)
        ki = lax.bitcast_convert_type(s, jnp.int32)
        keys[sl] = jnp.where(ki >= 0, ki, ki ^ jnp.int32(0x7FFFFFFF))
        scs[sl] = s
        bx1[sl] = x1
        by1[sl] = y1
        bx2[sl] = x2
        by2[sl] = y2
        bar[sl] = ws_ * hs_
        return c

    lax.fori_loop(0, VPT, dec, 0)
    plsc.subcore_barrier()

    def exch_lane(c_local, stamp):
        sti[...] = jnp.where(io == 0, c_local, jnp.where(io == 1, stamp, 0))
        pltpu.sync_copy(sti, cnt_sh.at[pl.ds(sid * 128, 16)])
        plsc.subcore_barrier()

        def cond(ok):
            return jnp.logical_not(ok)

        def body(ok):
            pltpu.sync_copy(cnt_sh, cntv)
            stamps = plsc.load_gather(cntv, [io * 128 + 1])
            return jnp.sum(jnp.where(stamps == stamp, 1, 0)) == NS

        lax.while_loop(cond, body, jnp.bool_(False))
        plsc.subcore_barrier()
        return plsc.load_gather(cntv, [io * 128])

    def exch(c_local, stamp):
        return jnp.sum(exch_lane(c_local, stamp))

    def count_pred(pred):
        def cb(j, acc):
            return acc + jnp.where(pred(j), 1, 0)
        return jnp.sum(lax.fori_loop(0, VPT, cb, zi))

    def count_ge(mid):
        return count_pred(lambda j: keys[pl.ds(j * 16, 16)] >= mid)

    c_pos = exch(count_ge(jnp.int32(1)), jnp.int32(1))

    def bs1(i, lohi):
        lo, hi = lohi
        mid = lo + ((hi - lo + 1) >> 1)
        c = exch(count_ge(mid), i + 2)
        big = c >= PRE_NMS_TOPN
        return (jnp.where(big, mid, lo), jnp.where(big, hi, mid - 1))

    lo, _ = lax.fori_loop(0, 30, bs1, (jnp.int32(1), jnp.int32(_KEY_HI)))
    tkey = jnp.where(c_pos >= PRE_NMS_TOPN, lo, jnp.int32(_KEY_NEG1E9))

    c_gt = exch(count_pred(lambda j: keys[pl.ds(j * 16, 16)] > tkey),
                jnp.int32(33))

    def count_eq_le(jmax):
        def pred(j):
            k = keys[pl.ds(j * 16, 16)]
            gi = base + j * 16 + io
            return (k == tkey) & (gi <= jmax)
        return count_pred(pred)

    def bs2(i, lohi):
        lo2, hi2 = lohi
        mid = (lo2 + hi2) >> 1
        h = c_gt + exch(count_eq_le(mid), i + 34)
        ok = h >= PRE_NMS_TOPN
        return (jnp.where(ok, lo2, mid + 1), jnp.where(ok, mid, hi2))

    idx_t, _ = lax.fori_loop(0, 15, bs2, (jnp.int32(0), jnp.int32(N_PAD - 1)))

    @pl.when(sid == 0)
    def _():
        st = jnp.where(io == 0, c_pos.astype(jnp.float32), jnp.float32(0.0))
        st = jnp.where(io == 1, (tkey >> 12).astype(jnp.float32), st)
        st = jnp.where(io == 2, (tkey & 0xFFF).astype(jnp.float32), st)
        st = jnp.where(io == 3, c_gt.astype(jnp.float32), st)
        st = jnp.where(io == 4, idx_t.astype(jnp.float32), st)
        rois[pl.ds(300 * 16, 16)] = st

    def cmp_(j, off):
        sl = pl.ds(j * 16, 16)
        k = keys[sl]
        gi = base + j * 16 + io
        elig = (k > tkey) | ((k == tkey) & (gi <= idx_t))
        dsl = pl.ds(off, 16)
        plsc.store_compressed(cs.at[dsl], scs[sl], mask=elig)
        plsc.store_compressed(cx1.at[dsl], bx1[sl], mask=elig)
        plsc.store_compressed(cy1.at[dsl], by1[sl], mask=elig)
        plsc.store_compressed(cx2.at[dsl], bx2[sl], mask=elig)
        plsc.store_compressed(cy2.at[dsl], by2[sl], mask=elig)
        plsc.store_compressed(car.at[dsl], bar[sl], mask=elig)
        plsc.store_compressed(cgi.at[dsl], gi, mask=elig)
        return off + jnp.max(plsc.all_reduce_population_count(elig))

    m = lax.fori_loop(0, VPT, cmp_, jnp.int32(0))
    cs[pl.ds(m, 16)] = jnp.full((16,), NEGB, jnp.float32)
    mvr = (m + 15) >> 4

    def nms(k, c):
        stampf = (k + 1).astype(jnp.float32)

        def am(j, st):
            bs_, bj = st
            s = cs[pl.ds(j * 16, 16)]
            upd = s > bs_
            return (jnp.where(upd, s, bs_), jnp.where(upd, j, bj))

        bs_, bj = lax.fori_loop(
            0, mvr, am, (jnp.full((16,), NEGB, jnp.float32), zi))
        smax = jnp.max(bs_)
        lidx = jnp.min(jnp.where(bs_ == smax, bj * 16 + io,
                                 jnp.int32(1 << 30)))
        fiv = zi + lidx
        xv = plsc.load_gather(cx1, [fiv])
        yv = plsc.load_gather(cy1, [fiv])
        x2v = plsc.load_gather(cx2, [fiv])
        y2v = plsc.load_gather(cy2, [fiv])
        av = plsc.load_gather(car, [fiv])
        gixf = jnp.max(plsc.load_gather(cgi, [fiv])).astype(jnp.float32)
        rec = jnp.where(io == 0, smax, jnp.float32(0.0))
        rec = jnp.where(io == 1, gixf, rec)
        rec = jnp.where(io == 2, xv, rec)
        rec = jnp.where(io == 3, yv, rec)
        rec = jnp.where(io == 4, x2v, rec)
        rec = jnp.where(io == 5, y2v, rec)
        rec = jnp.where(io == 6, av, rec)
        rec = jnp.where(io == 7, stampf, rec)
        stf[...] = rec
        pltpu.sync_copy(stf, rec_sh.at[pl.ds(sid * 128, 16)])
        plsc.subcore_barrier()

        def cond(ok):
            return jnp.logical_not(ok)

        def body(ok):
            pltpu.sync_copy(rec_sh, recv)
            stamps = plsc.load_gather(recv, [io * 128 + 7])
            return jnp.sum(jnp.where(stamps == stampf, 1, 0)) == NS

        lax.while_loop(cond, body, jnp.bool_(False))
        plsc.subcore_barrier()

        sa = plsc.load_gather(recv, [io * 128])
        ga = plsc.load_gather(recv, [io * 128 + 1])
        gmax = jnp.max(sa)
        valid = gmax > VALID_T
        gwin = jnp.min(jnp.where(sa == gmax, ga, jnp.float32(1e18)))
        wslot = (gwin.astype(jnp.int32) // PER) * 128
        xw = plsc.load_gather(recv, [zi + wslot + 2])
        yw = plsc.load_gather(recv, [zi + wslot + 3])
        xw2 = plsc.load_gather(recv, [zi + wslot + 4])
        y2w_ = plsc.load_gather(recv, [zi + wslot + 5])
        yw2 = y2w_
        arw = plsc.load_gather(recv, [zi + wslot + 6])

        @pl.when(valid)
        def _():
            def up(j, cc):
                sl = pl.ds(j * 16, 16)
                xx1 = jnp.maximum(xw, cx1[sl])
                yy1 = jnp.maximum(yw, cy1[sl])
                xx2 = jnp.minimum(xw2, cx2[sl])
                yy2 = jnp.minimum(yw2, cy2[sl])
                iw = jnp.maximum(0.0, xx2 - xx1 + 1.0)
                ih = jnp.maximum(0.0, yy2 - yy1 + 1.0)
                inter = iw * ih
                supp = inter * jnp.float32(1.7) > \
                    jnp.float32(0.7) * (arw + car[sl])
                cs[sl] = jnp.where(supp, NEGB, cs[sl])
                return cc

            lax.fori_loop(0, mvr, up, 0)

        @pl.when(valid & (sid == 0))
        def _():
            row = jnp.where(io == 1, xw, jnp.float32(0.0))
            row = jnp.where(io == 2, yw, row)
            row = jnp.where(io == 3, xw2, row)
            row = jnp.where(io == 4, yw2, row)
            plsc.store_scatter(rois, [k * 16 + io], row, mask=io < 5)

        return c

    lax.fori_loop(0, POST_NMS_TOPN, nms, 0)

    @pl.when(sid == 0)
    def _():
        pltpu.sync_copy(rois, out_h)


def _run_full(rpn_cls_score, rpn_bbox_pred):
    cls2 = rpn_cls_score.reshape(-1, 2)
    d = rpn_bbox_pred.reshape(-1, 4)
    pad = N_PAD - N_REAL

    def padded(col):
        return jnp.pad(col, (0, pad))

    args = [padded(cls2[:, 0]), padded(cls2[:, 1]),
            padded(d[:, 0]), padded(d[:, 1]), padded(d[:, 2]),
            padded(d[:, 3]),
            jnp.asarray(_AW), jnp.asarray(_AH),
            jnp.asarray(_ACX), jnp.asarray(_ACY)]

    mesh = plsc.VectorSubcoreMesh(core_axis_name="c", subcore_axis_name="s",
                                  num_cores=1, num_subcores=NS)
    f32, i32 = jnp.float32, jnp.int32
    scratch = [
        pltpu.VMEM((PER,), f32),
        pltpu.VMEM((PER,), f32),
        pltpu.VMEM((PER,), f32),
        pltpu.VMEM((PER,), f32),
        pltpu.VMEM((PER,), f32),
        pltpu.VMEM((PER,), f32),
        pltpu.VMEM((PER,), f32),
        pltpu.VMEM((PER,), f32),
        pltpu.VMEM((PER,), f32),
        pltpu.VMEM((PER,), f32),
        pltpu.VMEM((PER,), i32),
        pltpu.VMEM((PER,), f32),
        pltpu.VMEM((PER,), f32),
        pltpu.VMEM((PER,), f32),
        pltpu.VMEM((PER,), f32),
        pltpu.VMEM((PER,), f32),
        pltpu.VMEM((PER,), f32),
        pltpu.VMEM((PER + 16,), f32),
        pltpu.VMEM((PER + 16,), f32),
        pltpu.VMEM((PER + 16,), f32),
        pltpu.VMEM((PER + 16,), f32),
        pltpu.VMEM((PER + 16,), f32),
        pltpu.VMEM((PER + 16,), f32),
        pltpu.VMEM((PER + 16,), i32),
        pltpu.VMEM((16,), f32),
        pltpu.VMEM((16,), i32),
        pltpu.VMEM((2048,), f32),
        pltpu.VMEM((2048,), i32),
        pltpu.VMEM((OUT_ROWS * 16,), f32),
        pltpu.VMEM_SHARED((2048,), i32),
        pltpu.VMEM_SHARED((2048,), f32),
    ]
    out = pl.kernel(
        _body,
        out_type=jax.ShapeDtypeStruct((OUT_ROWS * 16,), jnp.float32),
        mesh=mesh,
        scratch_types=scratch,
        compiler_params=pltpu.CompilerParams(needs_layout_passes=False),
    )(*args)
    return out.reshape(OUT_ROWS, 16)


def kernel(rpn_cls_score, rpn_bbox_pred):
    return _run_full(rpn_cls_score, rpn_bbox_pred)[:POST_NMS_TOPN, :5]
